# baseline (device time: 48922 ns/iter reference)
import jax
import jax.numpy as jnp
from jax import lax
from jax.experimental import pallas as pl
from jax.experimental.pallas import tpu as pltpu

N_Z = 4


def kernel(x, dy):
    d_per, m = x.shape
    _, n = dy.shape
    chunk = m // N_Z

    def body(x_ref, dy_ref, out_ref, acc_ref, comm_ref, send_sems, recv_sems):
        my_x = lax.axis_index("x")
        my_y = lax.axis_index("y")
        my_z = lax.axis_index("z")
        right = (my_z + 1) % N_Z
        left = (my_z - 1) % N_Z

        barrier_sem = pltpu.get_barrier_semaphore()
        for nbr in (left, right):
            pl.semaphore_signal(
                barrier_sem, inc=1,
                device_id=(my_x, my_y, nbr),
                device_id_type=pl.DeviceIdType.MESH,
            )
        pl.semaphore_wait(barrier_sem, 2)

        acc_ref[:, :] = lax.dot_general(
            x_ref[:, :], dy_ref[:, :],
            dimension_numbers=(((0,), (0,)), ((), ())),
            preferred_element_type=jnp.float32,
        )

        for h in range(N_Z - 1):
            send_chunk = (my_z - 1 - h) % N_Z
            rdma = pltpu.make_async_remote_copy(
                src_ref=acc_ref.at[pl.ds(send_chunk * chunk, chunk), :],
                dst_ref=comm_ref.at[h],
                send_sem=send_sems.at[h],
                recv_sem=recv_sems.at[h],
                device_id=(my_x, my_y, right),
                device_id_type=pl.DeviceIdType.MESH,
            )
            rdma.start()
            rdma.wait()

            recv_chunk = (my_z - 2 - h) % N_Z
            sl = pl.ds(recv_chunk * chunk, chunk)
            acc_ref[sl, :] = acc_ref[sl, :] + comm_ref[h]

        out_ref[:, :] = acc_ref[pl.ds(my_z * chunk, chunk), :]

    return pl.pallas_call(
        body,
        out_shape=jax.ShapeDtypeStruct((chunk, n), jnp.float32),
        in_specs=[
            pl.BlockSpec(memory_space=pltpu.VMEM),
            pl.BlockSpec(memory_space=pltpu.VMEM),
        ],
        out_specs=pl.BlockSpec(memory_space=pltpu.VMEM),
        scratch_shapes=[
            pltpu.VMEM((m, n), jnp.float32),
            pltpu.VMEM((N_Z - 1, chunk, n), jnp.float32),
            pltpu.SemaphoreType.DMA((N_Z - 1,)),
            pltpu.SemaphoreType.DMA((N_Z - 1,)),
        ],
        compiler_params=pltpu.CompilerParams(collective_id=0),
    )(x, dy)


# device time: 35816 ns/iter; 1.3659x vs baseline; 1.3659x over previous
import jax
import jax.numpy as jnp
from jax import lax
from jax.experimental import pallas as pl
from jax.experimental.pallas import tpu as pltpu

N_Z = 4
MESH = pl.DeviceIdType.MESH


def kernel(x, dy):
    d_per, m = x.shape
    _, n = dy.shape
    chunk = m // N_Z
    qw = n // 4

    def body(x_ref, dy_ref, out_ref, acc_ref, pbuf_ref, mbuf_ref,
             p_send, p_recv, m_send, m_recv, ag_send, ag_recv):
        my_x = lax.axis_index("x")
        my_y = lax.axis_index("y")
        my_z = lax.axis_index("z")
        q = 2 * my_x + my_y
        qy = 2 * my_x + (1 - my_y)

        row = lambda c: pl.ds(c * chunk, chunk)
        qcol = lambda i: pl.ds(i * qw, qw)

        barrier_sem = pltpu.get_barrier_semaphore()
        pl.semaphore_signal(barrier_sem, inc=1,
                            device_id=(1 - my_x, my_y, my_z),
                            device_id_type=MESH)
        pl.semaphore_signal(barrier_sem, inc=1,
                            device_id=(my_x, 1 - my_y, my_z),
                            device_id_type=MESH)

        @pl.when(my_z > 0)
        def _():
            pl.semaphore_signal(barrier_sem, inc=1,
                                device_id=(my_x, my_y, my_z - 1),
                                device_id_type=MESH)

        @pl.when(my_z < N_Z - 1)
        def _():
            pl.semaphore_signal(barrier_sem, inc=1,
                                device_id=(my_x, my_y, my_z + 1),
                                device_id_type=MESH)

        interior = (my_z > 0) & (my_z < N_Z - 1)

        @pl.when(interior)
        def _():
            pl.semaphore_wait(barrier_sem, 4)

        @pl.when(jnp.logical_not(interior))
        def _():
            pl.semaphore_wait(barrier_sem, 3)

        acc_ref[:, :] = lax.dot_general(
            x_ref[:, :], dy_ref[:, qcol(q)],
            dimension_numbers=(((0,), (0,)), ((), ())),
            preferred_element_type=jnp.float32,
        )

        up_z = jnp.minimum(my_z + 1, N_Z - 1)
        dn_z = jnp.maximum(my_z - 1, 0)
        p_rd = {
            c: pltpu.make_async_remote_copy(
                src_ref=pbuf_ref.at[row(c), :],
                dst_ref=pbuf_ref.at[row(c), :],
                send_sem=p_send.at[c], recv_sem=p_recv.at[c],
                device_id=(my_x, my_y, up_z), device_id_type=MESH,
            )
            for c in (3, 2, 1)
        }
        m_rd = {
            c: pltpu.make_async_remote_copy(
                src_ref=mbuf_ref.at[row(c), :],
                dst_ref=mbuf_ref.at[row(c), :],
                send_sem=m_send.at[c], recv_sem=m_recv.at[c],
                device_id=(my_x, my_y, dn_z), device_id_type=MESH,
            )
            for c in (0, 1, 2)
        }

        for c in (3, 2, 1):
            @pl.when(my_z == 0)
            def _(c=c):
                pbuf_ref[row(c), :] = acc_ref[row(c), :]
                p_rd[c].start()

        for c in (0, 1, 2):
            @pl.when(my_z == N_Z - 1)
            def _(c=c):
                mbuf_ref[row(c), :] = acc_ref[row(c), :]
                m_rd[c].start()

        for s in range(N_Z - 1):
            cp = N_Z - 1 - s
            cm = s

            @pl.when((my_z > 0) & (cp >= my_z))
            def _():
                p_rd[cp].wait_recv()

            @pl.when((my_z > 0) & (cp > my_z))
            def _():
                pbuf_ref[row(cp), :] = pbuf_ref[row(cp), :] + acc_ref[row(cp), :]
                p_rd[cp].start()

            @pl.when((my_z < N_Z - 1) & (cm <= my_z))
            def _():
                m_rd[cm].wait_recv()

            @pl.when((my_z < N_Z - 1) & (cm < my_z))
            def _():
                mbuf_ref[row(cm), :] = mbuf_ref[row(cm), :] + acc_ref[row(cm), :]
                m_rd[cm].start()

        out_ref[:, qcol(q)] = acc_ref[row(my_z), :]

        @pl.when(my_z > 0)
        def _():
            out_ref[:, qcol(q)] = out_ref[:, qcol(q)] + pbuf_ref[row(my_z), :]

        @pl.when(my_z < N_Z - 1)
        def _():
            out_ref[:, qcol(q)] = out_ref[:, qcol(q)] + mbuf_ref[row(my_z), :]

        rd_x = pltpu.make_async_remote_copy(
            src_ref=out_ref.at[:, qcol(q)], dst_ref=out_ref.at[:, qcol(q)],
            send_sem=ag_send.at[0], recv_sem=ag_recv.at[0],
            device_id=(1 - my_x, my_y, my_z), device_id_type=MESH,
        )
        rd_y = pltpu.make_async_remote_copy(
            src_ref=out_ref.at[:, qcol(q)], dst_ref=out_ref.at[:, qcol(q)],
            send_sem=ag_send.at[1], recv_sem=ag_recv.at[1],
            device_id=(my_x, 1 - my_y, my_z), device_id_type=MESH,
        )
        rd_x.start()
        rd_y.start()
        rd_y.wait_recv()
        rd_d = pltpu.make_async_remote_copy(
            src_ref=out_ref.at[:, qcol(qy)], dst_ref=out_ref.at[:, qcol(qy)],
            send_sem=ag_send.at[2], recv_sem=ag_recv.at[2],
            device_id=(1 - my_x, my_y, my_z), device_id_type=MESH,
        )
        rd_d.start()
        rd_x.wait_recv()
        rd_d.wait_recv()

        for c in (3, 2, 1):
            @pl.when(c > my_z)
            def _(c=c):
                p_rd[c].wait_send()
        for c in (0, 1, 2):
            @pl.when(c < my_z)
            def _(c=c):
                m_rd[c].wait_send()
        rd_x.wait_send()
        rd_y.wait_send()
        rd_d.wait_send()

    return pl.pallas_call(
        body,
        out_shape=jax.ShapeDtypeStruct((chunk, n), jnp.float32),
        in_specs=[
            pl.BlockSpec(memory_space=pltpu.VMEM),
            pl.BlockSpec(memory_space=pltpu.VMEM),
        ],
        out_specs=pl.BlockSpec(memory_space=pltpu.VMEM),
        scratch_shapes=[
            pltpu.VMEM((m, qw), jnp.float32),
            pltpu.VMEM((m, qw), jnp.float32),
            pltpu.VMEM((m, qw), jnp.float32),
            pltpu.SemaphoreType.DMA((N_Z,)),
            pltpu.SemaphoreType.DMA((N_Z,)),
            pltpu.SemaphoreType.DMA((N_Z,)),
            pltpu.SemaphoreType.DMA((N_Z,)),
            pltpu.SemaphoreType.DMA((3,)),
            pltpu.SemaphoreType.DMA((3,)),
        ],
        compiler_params=pltpu.CompilerParams(collective_id=0),
    )(x, dy)


# device time: 30504 ns/iter; 1.6038x vs baseline; 1.1741x over previous
import jax
import jax.numpy as jnp
from jax import lax
from jax.experimental import pallas as pl
from jax.experimental.pallas import tpu as pltpu

N_Z = 4
MESH = pl.DeviceIdType.MESH


def kernel(x, dy):
    d_per, m = x.shape
    _, n = dy.shape
    chunk = m // N_Z
    qw = n // 4
    hw = qw // 2

    def body(x_ref, dy_ref, out_ref, acc_ref, pbuf_ref, mbuf_ref,
             p_send, p_recv, m_send, m_recv, ag_send, ag_recv):
        my_x = lax.axis_index("x")
        my_y = lax.axis_index("y")
        my_z = lax.axis_index("z")
        q = 2 * my_x + my_y
        qx = 2 * (1 - my_x) + my_y
        qy = 2 * my_x + (1 - my_y)
        qd = 2 * (1 - my_x) + (1 - my_y)

        row = lambda c: pl.ds(c * chunk, chunk)
        qcol = lambda i: pl.ds(i * qw, qw)
        hcol = lambda i, h: pl.ds(i * qw + h * hw, hw)

        acc_ref[:, :] = lax.dot_general(
            x_ref[:, :], dy_ref[:, qcol(q)],
            dimension_numbers=(((0,), (0,)), ((), ())),
            preferred_element_type=jnp.float32,
        )

        barrier_sem = pltpu.get_barrier_semaphore()
        pl.semaphore_signal(barrier_sem, inc=1,
                            device_id=(1 - my_x, my_y, my_z),
                            device_id_type=MESH)
        pl.semaphore_signal(barrier_sem, inc=1,
                            device_id=(my_x, 1 - my_y, my_z),
                            device_id_type=MESH)

        @pl.when(my_z > 0)
        def _():
            pl.semaphore_signal(barrier_sem, inc=1,
                                device_id=(my_x, my_y, my_z - 1),
                                device_id_type=MESH)

        @pl.when(my_z < N_Z - 1)
        def _():
            pl.semaphore_signal(barrier_sem, inc=1,
                                device_id=(my_x, my_y, my_z + 1),
                                device_id_type=MESH)

        interior = (my_z > 0) & (my_z < N_Z - 1)

        @pl.when(interior)
        def _():
            pl.semaphore_wait(barrier_sem, 4)

        @pl.when(jnp.logical_not(interior))
        def _():
            pl.semaphore_wait(barrier_sem, 3)

        up_z = jnp.minimum(my_z + 1, N_Z - 1)
        dn_z = jnp.maximum(my_z - 1, 0)
        p_rd = {
            c: pltpu.make_async_remote_copy(
                src_ref=pbuf_ref.at[row(c), :],
                dst_ref=pbuf_ref.at[row(c), :],
                send_sem=p_send.at[c], recv_sem=p_recv.at[c],
                device_id=(my_x, my_y, up_z), device_id_type=MESH,
            )
            for c in (3, 2, 1)
        }
        m_rd = {
            c: pltpu.make_async_remote_copy(
                src_ref=mbuf_ref.at[row(c), :],
                dst_ref=mbuf_ref.at[row(c), :],
                send_sem=m_send.at[c], recv_sem=m_recv.at[c],
                device_id=(my_x, my_y, dn_z), device_id_type=MESH,
            )
            for c in (0, 1, 2)
        }
        inj_p = {
            c: pltpu.make_async_remote_copy(
                src_ref=acc_ref.at[row(c), :],
                dst_ref=pbuf_ref.at[row(c), :],
                send_sem=p_send.at[c], recv_sem=p_recv.at[c],
                device_id=(my_x, my_y, up_z), device_id_type=MESH,
            )
            for c in (3, 2, 1)
        }
        inj_m = {
            c: pltpu.make_async_remote_copy(
                src_ref=acc_ref.at[row(c), :],
                dst_ref=mbuf_ref.at[row(c), :],
                send_sem=m_send.at[c], recv_sem=m_recv.at[c],
                device_id=(my_x, my_y, dn_z), device_id_type=MESH,
            )
            for c in (0, 1, 2)
        }

        for c in (3, 2, 1):
            @pl.when(my_z == 0)
            def _(c=c):
                inj_p[c].start()

        for c in (0, 1, 2):
            @pl.when(my_z == N_Z - 1)
            def _(c=c):
                inj_m[c].start()

        def emit_plus(s, extra):
            cp = N_Z - 1 - s

            @pl.when(extra & (my_z > 0) & (cp >= my_z))
            def _():
                p_rd[cp].wait_recv()

            @pl.when(extra & (my_z > 0) & (cp > my_z))
            def _():
                pbuf_ref[row(cp), :] = pbuf_ref[row(cp), :] + acc_ref[row(cp), :]
                p_rd[cp].start()

        def emit_minus(s, extra):
            cm = s

            @pl.when(extra & (my_z < N_Z - 1) & (cm <= my_z))
            def _():
                m_rd[cm].wait_recv()

            @pl.when(extra & (my_z < N_Z - 1) & (cm < my_z))
            def _():
                mbuf_ref[row(cm), :] = mbuf_ref[row(cm), :] + acc_ref[row(cm), :]
                m_rd[cm].start()

        low = my_z <= 1
        high = jnp.logical_not(low)
        for s in range(N_Z - 1):
            emit_plus(s, low)
            emit_minus(s, low)
            emit_minus(s, high)
            emit_plus(s, high)

        out_ref[:, qcol(q)] = acc_ref[row(my_z), :]

        @pl.when(my_z > 0)
        def _():
            out_ref[:, qcol(q)] = out_ref[:, qcol(q)] + pbuf_ref[row(my_z), :]

        @pl.when(my_z < N_Z - 1)
        def _():
            out_ref[:, qcol(q)] = out_ref[:, qcol(q)] + mbuf_ref[row(my_z), :]

        rd_x = pltpu.make_async_remote_copy(
            src_ref=out_ref.at[:, qcol(q)], dst_ref=out_ref.at[:, qcol(q)],
            send_sem=ag_send.at[0], recv_sem=ag_recv.at[0],
            device_id=(1 - my_x, my_y, my_z), device_id_type=MESH,
        )
        rd_y = pltpu.make_async_remote_copy(
            src_ref=out_ref.at[:, qcol(q)], dst_ref=out_ref.at[:, qcol(q)],
            send_sem=ag_send.at[1], recv_sem=ag_recv.at[1],
            device_id=(my_x, 1 - my_y, my_z), device_id_type=MESH,
        )
        rd_x.start()
        rd_y.start()
        rd_dx = pltpu.make_async_remote_copy(
            src_ref=out_ref.at[:, hcol(qy, 0)], dst_ref=out_ref.at[:, hcol(qy, 0)],
            send_sem=ag_send.at[2], recv_sem=ag_recv.at[2],
            device_id=(1 - my_x, my_y, my_z), device_id_type=MESH,
        )
        rd_dy = pltpu.make_async_remote_copy(
            src_ref=out_ref.at[:, hcol(qx, 1)], dst_ref=out_ref.at[:, hcol(qx, 1)],
            send_sem=ag_send.at[3], recv_sem=ag_recv.at[3],
            device_id=(my_x, 1 - my_y, my_z), device_id_type=MESH,
        )
        rd_y.wait_recv()
        rd_dx.start()
        rd_x.wait_recv()
        rd_dy.start()
        rd_dx.wait_recv()
        rd_dy.wait_recv()

        for c in (3, 2, 1):
            @pl.when(my_z == 0)
            def _(c=c):
                inj_p[c].wait_send()

            @pl.when((my_z > 0) & (c > my_z))
            def _(c=c):
                p_rd[c].wait_send()
        for c in (0, 1, 2):
            @pl.when(my_z == N_Z - 1)
            def _(c=c):
                inj_m[c].wait_send()

            @pl.when((my_z < N_Z - 1) & (c < my_z))
            def _(c=c):
                m_rd[c].wait_send()
        rd_x.wait_send()
        rd_y.wait_send()
        rd_dx.wait_send()
        rd_dy.wait_send()

    return pl.pallas_call(
        body,
        out_shape=jax.ShapeDtypeStruct((chunk, n), jnp.float32),
        in_specs=[
            pl.BlockSpec(memory_space=pltpu.VMEM),
            pl.BlockSpec(memory_space=pltpu.VMEM),
        ],
        out_specs=pl.BlockSpec(memory_space=pltpu.VMEM),
        scratch_shapes=[
            pltpu.VMEM((m, qw), jnp.float32),
            pltpu.VMEM((m, qw), jnp.float32),
            pltpu.VMEM((m, qw), jnp.float32),
            pltpu.SemaphoreType.DMA((N_Z,)),
            pltpu.SemaphoreType.DMA((N_Z,)),
            pltpu.SemaphoreType.DMA((N_Z,)),
            pltpu.SemaphoreType.DMA((N_Z,)),
            pltpu.SemaphoreType.DMA((4,)),
            pltpu.SemaphoreType.DMA((4,)),
        ],
        compiler_params=pltpu.CompilerParams(collective_id=0),
    )(x, dy)


# device time: 24883 ns/iter; 1.9661x vs baseline; 1.2259x over previous
import os

import jax
import jax.numpy as jnp
from jax import lax
from jax.experimental import pallas as pl
from jax.experimental.pallas import tpu as pltpu

N_Z = 4
MESH = pl.DeviceIdType.MESH

_SKIP_AG = os.environ.get("SKIP_AG") == "1"
_SKIP_CHAIN = os.environ.get("SKIP_CHAIN") == "1"


def kernel(x, dy):
    d_per, m = x.shape
    _, n = dy.shape
    chunk = m // N_Z
    qw = n // 4
    hw = qw // 2

    def body(x_ref, dy_ref, out_ref, acc_ref, pbuf_ref, mbuf_ref, qbuf_ref,
             x_vmem, dyq_vmem, p_send, p_recv, m_send, m_recv,
             ag_send, ag_recv, in_sems, out_sem):
        my_x = lax.axis_index("x")
        my_y = lax.axis_index("y")
        my_z = lax.axis_index("z")
        q = 2 * my_x + my_y

        row = lambda c: pl.ds(c * chunk, chunk)
        qcol = lambda i: pl.ds(i * qw, qw)
        lh = lambda h: pl.ds(h * hw, hw)

        cp_x = pltpu.make_async_copy(x_ref, x_vmem, in_sems.at[0])
        cp_dy = pltpu.make_async_copy(
            dy_ref.at[:, qcol(q)], dyq_vmem, in_sems.at[1])
        cp_x.start()
        cp_dy.start()

        barrier_sem = pltpu.get_barrier_semaphore()
        pl.semaphore_signal(barrier_sem, inc=1,
                            device_id=(1 - my_x, my_y, my_z),
                            device_id_type=MESH)
        pl.semaphore_signal(barrier_sem, inc=1,
                            device_id=(my_x, 1 - my_y, my_z),
                            device_id_type=MESH)
        pl.semaphore_signal(barrier_sem, inc=1,
                            device_id=(1 - my_x, 1 - my_y, my_z),
                            device_id_type=MESH)

        @pl.when(my_z > 0)
        def _():
            pl.semaphore_signal(barrier_sem, inc=1,
                                device_id=(my_x, my_y, my_z - 1),
                                device_id_type=MESH)

        @pl.when(my_z < N_Z - 1)
        def _():
            pl.semaphore_signal(barrier_sem, inc=1,
                                device_id=(my_x, my_y, my_z + 1),
                                device_id_type=MESH)

        cp_x.wait()
        cp_dy.wait()
        acc_ref[:, :] = lax.dot_general(
            x_vmem[:, :], dyq_vmem[:, :],
            dimension_numbers=(((0,), (0,)), ((), ())),
            preferred_element_type=jnp.float32,
        )

        interior = (my_z > 0) & (my_z < N_Z - 1)

        @pl.when(interior)
        def _():
            pl.semaphore_wait(barrier_sem, 5)

        @pl.when(jnp.logical_not(interior))
        def _():
            pl.semaphore_wait(barrier_sem, 4)

        up_z = jnp.minimum(my_z + 1, N_Z - 1)
        dn_z = jnp.maximum(my_z - 1, 0)

        def chain_rd(buf, c, h, sems_s, sems_r, dst_z, src=None):
            return pltpu.make_async_remote_copy(
                src_ref=(buf if src is None else src).at[row(c), lh(h)],
                dst_ref=buf.at[row(c), lh(h)],
                send_sem=sems_s.at[c, h], recv_sem=sems_r.at[c, h],
                device_id=(my_x, my_y, dst_z), device_id_type=MESH,
            )

        p_rd = {(c, h): chain_rd(pbuf_ref, c, h, p_send, p_recv, up_z)
                for c in (3, 2, 1) for h in (0, 1)}
        m_rd = {(c, h): chain_rd(mbuf_ref, c, h, m_send, m_recv, dn_z)
                for c in (0, 1, 2) for h in (0, 1)}
        inj_p = {(c, h): chain_rd(pbuf_ref, c, h, p_send, p_recv, up_z,
                                  src=acc_ref)
                 for c in (3, 2, 1) for h in (0, 1)}
        inj_m = {(c, h): chain_rd(mbuf_ref, c, h, m_send, m_recv, dn_z,
                                  src=acc_ref)
                 for c in (0, 1, 2) for h in (0, 1)}

        for c in (3, 2, 1) if not _SKIP_CHAIN else ():
            @pl.when(my_z == 0)
            def _(c=c):
                inj_p[c, 0].start()
                inj_p[c, 1].start()

        for c in (0, 1, 2) if not _SKIP_CHAIN else ():
            @pl.when(my_z == N_Z - 1)
            def _(c=c):
                inj_m[c, 0].start()
                inj_m[c, 1].start()

        def emit_plus(s, extra):
            cp = N_Z - 1 - s
            for h in (0, 1):
                @pl.when(extra & (my_z > 0) & (cp >= my_z))
                def _(h=h):
                    p_rd[cp, h].wait_recv()

                @pl.when(extra & (my_z > 0) & (cp > my_z))
                def _(h=h):
                    pbuf_ref[row(cp), lh(h)] = (
                        pbuf_ref[row(cp), lh(h)] + acc_ref[row(cp), lh(h)]
                    )
                    p_rd[cp, h].start()

        def emit_minus(s, extra):
            cm = s
            for h in (0, 1):
                @pl.when(extra & (my_z < N_Z - 1) & (cm <= my_z))
                def _(h=h):
                    m_rd[cm, h].wait_recv()

                @pl.when(extra & (my_z < N_Z - 1) & (cm < my_z))
                def _(h=h):
                    mbuf_ref[row(cm), lh(h)] = (
                        mbuf_ref[row(cm), lh(h)] + acc_ref[row(cm), lh(h)]
                    )
                    m_rd[cm, h].start()

        low = my_z <= 1
        high = jnp.logical_not(low)
        for s in range(N_Z - 1) if not _SKIP_CHAIN else ():
            emit_plus(s, low)
            emit_minus(s, low)
            emit_minus(s, high)
            emit_plus(s, high)

        qbuf_ref[:, :] = acc_ref[row(my_z), :]

        if not _SKIP_CHAIN:
            @pl.when(my_z > 0)
            def _():
                qbuf_ref[:, :] = qbuf_ref[:, :] + pbuf_ref[row(my_z), :]

            @pl.when(my_z < N_Z - 1)
            def _():
                qbuf_ref[:, :] = qbuf_ref[:, :] + mbuf_ref[row(my_z), :]

        cp_out = pltpu.make_async_copy(
            qbuf_ref, out_ref.at[:, qcol(q)], out_sem)
        cp_out.start()

        if not _SKIP_AG:
            def ag_rd(idx, dev):
                return pltpu.make_async_remote_copy(
                    src_ref=qbuf_ref,
                    dst_ref=out_ref.at[:, qcol(q)],
                    send_sem=ag_send.at[idx], recv_sem=ag_recv.at[idx],
                    device_id=dev, device_id_type=MESH,
                )

            rd_dg = ag_rd(0, (1 - my_x, 1 - my_y, my_z))
            rd_x = ag_rd(1, (1 - my_x, my_y, my_z))
            rd_y = ag_rd(2, (my_x, 1 - my_y, my_z))
            rd_dg.start()
            rd_x.start()
            rd_y.start()
            rd_x.wait_recv()
            rd_y.wait_recv()
            rd_dg.wait_recv()
        cp_out.wait()

        for c in (3, 2, 1) if not _SKIP_CHAIN else ():
            for h in (0, 1):
                @pl.when(my_z == 0)
                def _(c=c, h=h):
                    inj_p[c, h].wait_send()

                @pl.when((my_z > 0) & (c > my_z))
                def _(c=c, h=h):
                    p_rd[c, h].wait_send()
        for c in (0, 1, 2) if not _SKIP_CHAIN else ():
            for h in (0, 1):
                @pl.when(my_z == N_Z - 1)
                def _(c=c, h=h):
                    inj_m[c, h].wait_send()

                @pl.when((my_z < N_Z - 1) & (c < my_z))
                def _(c=c, h=h):
                    m_rd[c, h].wait_send()
        if not _SKIP_AG:
            for rd in (rd_dg, rd_x, rd_y):
                rd.wait_send()

    return pl.pallas_call(
        body,
        out_shape=jax.ShapeDtypeStruct((chunk, n), jnp.float32),
        in_specs=[
            pl.BlockSpec(memory_space=pl.ANY),
            pl.BlockSpec(memory_space=pl.ANY),
        ],
        out_specs=pl.BlockSpec(memory_space=pltpu.MemorySpace.HBM),
        scratch_shapes=[
            pltpu.VMEM((m, qw), jnp.float32),
            pltpu.VMEM((m, qw), jnp.float32),
            pltpu.VMEM((m, qw), jnp.float32),
            pltpu.VMEM((chunk, qw), jnp.float32),
            pltpu.VMEM((d_per, m), jnp.float32),
            pltpu.VMEM((d_per, qw), jnp.float32),
            pltpu.SemaphoreType.DMA((N_Z, 2)),
            pltpu.SemaphoreType.DMA((N_Z, 2)),
            pltpu.SemaphoreType.DMA((N_Z, 2)),
            pltpu.SemaphoreType.DMA((N_Z, 2)),
            pltpu.SemaphoreType.DMA((3,)),
            pltpu.SemaphoreType.DMA((3,)),
            pltpu.SemaphoreType.DMA((2,)),
            pltpu.SemaphoreType.DMA(()),
        ],
        compiler_params=pltpu.CompilerParams(collective_id=0),
    )(
        pltpu.with_memory_space_constraint(x, pltpu.MemorySpace.HBM),
        pltpu.with_memory_space_constraint(dy, pltpu.MemorySpace.HBM),
    )
